# baseline (speedup 1.0000x reference)
import jax
import jax.numpy as jnp
from jax.experimental import pallas as pl
from jax.experimental.pallas import tpu as pltpu

N = 10000
D = 128
BM = 400
BH = 2000


def _hidden_body(x_ref, wt_ref, b_ref, h_ref):
    x = x_ref[...].astype(jnp.bfloat16)
    wt = wt_ref[...].astype(jnp.bfloat16)
    h = jnp.dot(x, wt, preferred_element_type=jnp.float32) + b_ref[...]
    h_ref[...] = h.astype(jnp.bfloat16)


def _matmul_body(a_ref, h_ref, o_ref):
    a = a_ref[...].astype(jnp.bfloat16)
    o_ref[...] = jnp.dot(a, h_ref[...], preferred_element_type=jnp.float32)


def kernel(input, A, W, b):
    wt = W.T
    b2 = b.reshape(1, D)

    hidden = pl.pallas_call(
        _hidden_body,
        grid=(N // BH,),
        in_specs=[
            pl.BlockSpec((BH, D), lambda i: (i, 0)),
            pl.BlockSpec((D, D), lambda i: (0, 0)),
            pl.BlockSpec((1, D), lambda i: (0, 0)),
        ],
        out_specs=pl.BlockSpec((BH, D), lambda i: (i, 0)),
        out_shape=jax.ShapeDtypeStruct((N, D), jnp.bfloat16),
        compiler_params=pltpu.CompilerParams(
            dimension_semantics=("parallel",),
        ),
    )(input, wt, b2)

    out = pl.pallas_call(
        _matmul_body,
        grid=(N // BM,),
        in_specs=[
            pl.BlockSpec((BM, N), lambda i: (i, 0)),
            pl.BlockSpec((N, D), lambda i: (0, 0)),
        ],
        out_specs=pl.BlockSpec((BM, D), lambda i: (i, 0)),
        out_shape=jax.ShapeDtypeStruct((N, D), jnp.float32),
        compiler_params=pltpu.CompilerParams(
            dimension_semantics=("parallel",),
        ),
    )(A, hidden)
    return out


# speedup vs baseline: 1.0324x; 1.0324x over previous
import jax
import jax.numpy as jnp
from jax.experimental import pallas as pl
from jax.experimental.pallas import tpu as pltpu

N = 10000
D = 128
BM = 400


def _body(x_ref, wt_ref, b_ref, a_ref, o_ref, h_ref):
    @pl.when(pl.program_id(0) == 0)
    def _compute_hidden():
        x = x_ref[...].astype(jnp.bfloat16)
        wt = wt_ref[...].astype(jnp.bfloat16)
        h = jnp.dot(x, wt, preferred_element_type=jnp.float32) + b_ref[...]
        h_ref[...] = h.astype(jnp.bfloat16)

    a = a_ref[...].astype(jnp.bfloat16)
    o_ref[...] = jnp.dot(a, h_ref[...], preferred_element_type=jnp.float32)


def kernel(input, A, W, b):
    wt = W.T
    b2 = b.reshape(1, D)

    out = pl.pallas_call(
        _body,
        grid=(N // BM,),
        in_specs=[
            pl.BlockSpec((N, D), lambda i: (0, 0)),
            pl.BlockSpec((D, D), lambda i: (0, 0)),
            pl.BlockSpec((1, D), lambda i: (0, 0)),
            pl.BlockSpec((BM, N), lambda i: (i, 0)),
        ],
        out_specs=pl.BlockSpec((BM, D), lambda i: (i, 0)),
        out_shape=jax.ShapeDtypeStruct((N, D), jnp.float32),
        scratch_shapes=[pltpu.VMEM((N, D), jnp.bfloat16)],
        compiler_params=pltpu.CompilerParams(
            dimension_semantics=("arbitrary",),
        ),
    )(input, wt, b2, A)
    return out
